# Initial kernel scaffold; baseline (speedup 1.0000x reference)
import functools

import jax
import jax.numpy as jnp
from jax import lax
from jax.experimental import pallas as pl
from jax.experimental.pallas import tpu as pltpu
from jax.experimental.pallas import tpu_sc as plsc

N = 10000
E = 320000
D = 128
H = 64
C = 40
K = 10

NP = 10240
ROWS_PER_TILE = NP // 16
RCH = 80
NCH = ROWS_PER_TILE // RCH
EPT = 20480
ECH = 128
NECH = EPT // ECH
HD = D // 2


def _zero_rows(ref, nrows):
    z = jnp.zeros((16,), jnp.float32)
    @pl.loop(0, nrows)
    def _(i):
        for g in range(HD // 16):
            ref[i, pl.ds(g * 16, 16)] = z


def _sc_body(x_hbm, rowp_hbm, colp_hbm, temp_hbm,
             hid_hbm, u_hbm,
             row_v, col_v, gbuf, abuf, ubuf, sbuf, zbuf,
             deg_l, degbuf, disb, dis2b, invb, tempv,
             acc_sp, s_sp, degstage):
    c = lax.axis_index("c")
    tid = lax.axis_index("s")
    cnp = (c * NP).astype(jnp.int32)
    base = tid * ROWS_PER_TILE

    pltpu.sync_copy(rowp_hbm.at[tid], row_v)
    pltpu.sync_copy(colp_hbm.at[tid], col_v)
    pltpu.sync_copy(temp_hbm, tempv)
    cnp_v = jnp.full((16,), cnp, jnp.int32)
    @pl.loop(0, NECH)
    def _(j):
        for g in range(ECH // 16):
            sl = pl.ds(g * 16, 16)
            row_v[j, sl] = row_v[j, sl] + cnp_v

    _zero_rows(zbuf, RCH)
    z16 = jnp.zeros((16,), jnp.float32)
    @pl.loop(0, NP // 16)
    def _(i):
        deg_l[pl.ds(i * 16, 16)] = z16

    ones = jnp.full((16,), 1.0, jnp.float32)
    @pl.loop(0, NECH)
    def _(j):
        for g in range(ECH // 16):
            cv = col_v[j, pl.ds(g * 16, 16)]
            plsc.addupdate_scatter(deg_l, [cv], ones)
    pltpu.sync_copy(deg_l, degstage.at[tid])
    plsc.subcore_barrier()
    for t in range(16):
        pltpu.sync_copy(degstage.at[t, pl.ds(base, ROWS_PER_TILE)],
                        degbuf.at[t])

    half = jnp.full((16,), 0.5, jnp.float32)
    threehalf = jnp.full((16,), 1.5, jnp.float32)
    magic = jnp.full((16,), 0x5F3759DF, jnp.int32)
    one1 = jnp.full((16,), 1.0, jnp.float32)
    @pl.loop(0, ROWS_PER_TILE // 16)
    def _(i):
        sl = pl.ds(i * 16, 16)
        d = degbuf[0, sl]
        for t in range(1, 16):
            d = d + degbuf[t, sl]
        d = d + one1
        d2 = one1 / d
        y = plsc.bitcast(magic - (plsc.bitcast(d, jnp.int32) >> 1),
                         jnp.float32)
        hd = half * d
        for _it in range(3):
            y = y * (threehalf - hd * y * y)
        disb[sl] = y
        dis2b[sl] = d2
        invb[sl] = d * y

    t0v = tempv[0, :]
    @pl.loop(0, NCH)
    def _(jj):
        r0 = base + jj * RCH
        pltpu.sync_copy(x_hbm.at[pl.ds(cnp + r0, RCH)], ubuf)
        @pl.loop(0, RCH)
        def _(i):
            dv = disb[jj * RCH + i]
            for g in range(HD // 16):
                sl = pl.ds(g * 16, 16)
                un = dv * ubuf[i, sl]
                ubuf[i, sl] = un
                sbuf[i, sl] = t0v * un
        pltpu.sync_copy(ubuf, u_hbm.at[pl.ds(cnp + r0, RCH)])
        pltpu.sync_copy(sbuf, s_sp.at[pl.ds(r0, RCH)])
        pltpu.sync_copy(zbuf, acc_sp.at[pl.ds(r0, RCH)])
    plsc.subcore_barrier()

    for k in range(K):
        @pl.loop(0, NECH)
        def _(j):
            pltpu.sync_copy(u_hbm.at[row_v.at[j]], gbuf)
            pltpu.sync_copy(gbuf, acc_sp.at[col_v.at[j]], add=True)
        plsc.subcore_barrier()

        tkv = tempv[k + 1, :]
        @pl.loop(0, NCH)
        def _(jj):
            r0 = base + jj * RCH
            pltpu.sync_copy(acc_sp.at[pl.ds(r0, RCH)], abuf)
            pltpu.sync_copy(u_hbm.at[pl.ds(cnp + r0, RCH)], ubuf)
            pltpu.sync_copy(s_sp.at[pl.ds(r0, RCH)], sbuf)
            @pl.loop(0, RCH)
            def _(i):
                d2 = dis2b[jj * RCH + i]
                for g in range(HD // 16):
                    sl = pl.ds(g * 16, 16)
                    un = d2 * (abuf[i, sl] + ubuf[i, sl])
                    ubuf[i, sl] = un
                    sbuf[i, sl] = sbuf[i, sl] + tkv * un
            pltpu.sync_copy(ubuf, u_hbm.at[pl.ds(cnp + r0, RCH)])
            pltpu.sync_copy(sbuf, s_sp.at[pl.ds(r0, RCH)])
            pltpu.sync_copy(zbuf, acc_sp.at[pl.ds(r0, RCH)])
        plsc.subcore_barrier()

    @pl.loop(0, NCH)
    def _(jj):
        r0 = base + jj * RCH
        pltpu.sync_copy(s_sp.at[pl.ds(r0, RCH)], sbuf)
        @pl.loop(0, RCH)
        def _(i):
            iv = invb[jj * RCH + i]
            for g in range(HD // 16):
                sl = pl.ds(g * 16, 16)
                sbuf[i, sl] = iv * sbuf[i, sl]
        pltpu.sync_copy(sbuf, hid_hbm.at[pl.ds(cnp + r0, RCH)])


def _propagate(x_flat, rowp, colp, temp_b):
    mesh = plsc.VectorSubcoreMesh(core_axis_name="c", subcore_axis_name="s")
    f32 = jnp.float32
    kfn = pl.kernel(
        _sc_body,
        out_type=[
            jax.ShapeDtypeStruct((2 * NP, HD), f32),
            jax.ShapeDtypeStruct((2 * NP, HD), f32),
        ],
        mesh=mesh,
        scratch_types=[
            pltpu.VMEM((NECH, ECH), jnp.int32),
            pltpu.VMEM((NECH, ECH), jnp.int32),
            pltpu.VMEM((ECH, HD), f32),
            pltpu.VMEM((RCH, HD), f32),
            pltpu.VMEM((RCH, HD), f32),
            pltpu.VMEM((RCH, HD), f32),
            pltpu.VMEM((RCH, HD), f32),
            pltpu.VMEM((NP,), f32),
            pltpu.VMEM((16, ROWS_PER_TILE), f32),
            pltpu.VMEM((ROWS_PER_TILE,), f32),
            pltpu.VMEM((ROWS_PER_TILE,), f32),
            pltpu.VMEM((ROWS_PER_TILE,), f32),
            pltpu.VMEM((16, 16), f32),
            pltpu.VMEM_SHARED((NP, HD), f32),
            pltpu.VMEM_SHARED((NP, HD), f32),
            pltpu.VMEM_SHARED((16, NP), f32),
        ],
    )
    hid, _ = kfn(x_flat, rowp, colp, temp_b)
    return hid


def _mlp_body(h_ref, w1_ref, b1_ref, w2_ref, b2_ref, o_ref):
    z = jnp.dot(h_ref[...], w1_ref[...], preferred_element_type=jnp.float32)
    z = jnp.maximum(z + b1_ref[...], 0.0)
    lg = jnp.dot(z, w2_ref[...], preferred_element_type=jnp.float32)
    lg = lg + b2_ref[...]
    m = jnp.max(lg, axis=1, keepdims=True)
    s = jnp.log(jnp.sum(jnp.exp(lg - m), axis=1, keepdims=True))
    o_ref[...] = lg - m - s


def _mlp(hidden, W1, b1, W2, b2):
    BN = 1000
    grid = (N // BN,)
    return pl.pallas_call(
        _mlp_body,
        grid=grid,
        in_specs=[
            pl.BlockSpec((BN, D), lambda i: (i, 0)),
            pl.BlockSpec((D, H), lambda i: (0, 0)),
            pl.BlockSpec((1, H), lambda i: (0, 0)),
            pl.BlockSpec((H, C), lambda i: (0, 0)),
            pl.BlockSpec((1, C), lambda i: (0, 0)),
        ],
        out_specs=pl.BlockSpec((BN, C), lambda i: (i, 0)),
        out_shape=jax.ShapeDtypeStruct((N, C), jnp.float32),
    )(hidden, W1, b1.reshape(1, H), W2, b2.reshape(1, C))


@jax.jit
def kernel(x, edge_index, temp, W1, b1, W2, b2):
    row = edge_index[0]
    col = edge_index[1]
    pad = 16 * EPT - E
    rowp = jnp.concatenate([row, jnp.zeros((pad,), jnp.int32)])
    colp = jnp.concatenate([col, jnp.full((pad,), N, jnp.int32)])
    rowp = rowp.reshape(16, NECH, ECH)
    colp = colp.reshape(16, NECH, ECH)
    x0 = jnp.pad(x[:, :HD], ((0, NP - N), (0, 0)))
    x1 = jnp.pad(x[:, HD:], ((0, NP - N), (0, 0)))
    x_flat = jnp.concatenate([x0, x1], axis=0)
    temp_b = jnp.broadcast_to(jnp.pad(temp, (0, 16 - (K + 1)))[:, None],
                              (16, 16)).astype(jnp.float32)
    hid = _propagate(x_flat, rowp, colp, temp_b)
    hidden = jnp.concatenate([hid[:N], hid[NP:NP + N]], axis=1)
    return _mlp(hidden, W1, b1, W2, b2)


# speedup vs baseline: 5.8105x; 5.8105x over previous
import functools

import jax
import jax.numpy as jnp
from jax import lax
from jax.experimental import pallas as pl
from jax.experimental.pallas import tpu as pltpu
from jax.experimental.pallas import tpu_sc as plsc

N = 10000
E = 320000
D = 128
H = 64
C = 40
K = 10

NP = 10240
ROWS_PER_TILE = NP // 16
RCH = 40
NCH = ROWS_PER_TILE // RCH
EPT = 20480
ECH = 128
NECH = EPT // ECH
HD = D // 2


def _zero_rows(ref, nrows):
    z = jnp.zeros((16,), jnp.float32)
    @pl.loop(0, nrows)
    def _(i):
        for g in range(HD // 16):
            ref[i, pl.ds(g * 16, 16)] = z


def _sc_body(x_hbm, rowp_hbm, colp_hbm, temp_hbm,
             hid_hbm, u_hbm, s_hbm,
             row_v, col_v, gbuf, abuf, ubuf, sbuf, zbuf,
             dis2b, tempv,
             acc_sp):
    c = lax.axis_index("c")
    tid = lax.axis_index("s")
    cnp = (c * NP).astype(jnp.int32)
    base = tid * ROWS_PER_TILE

    ones = jnp.full((16,), 1.0, jnp.float32)
    half = jnp.full((16,), 0.5, jnp.float32)

    def babylonian_sqrt(d):
        y = half * (ones + d)
        for _it in range(12):
            y = half * (y + d / y)
        return y

    pltpu.sync_copy(rowp_hbm.at[tid], row_v)
    pltpu.sync_copy(colp_hbm.at[tid], col_v)
    pltpu.sync_copy(temp_hbm, tempv)
    cnp_v = jnp.full((16,), cnp, jnp.int32)
    @pl.loop(0, NECH)
    def _(j):
        for g in range(ECH // 16):
            sl = pl.ds(g * 16, 16)
            row_v[j, sl] = row_v[j, sl] + cnp_v

    _zero_rows(zbuf, RCH)

    @pl.loop(0, ECH)
    def _(i):
        for g in range(HD // 16):
            gbuf[i, pl.ds(g * 16, 16)] = ones
    @pl.loop(0, NCH)
    def _(jj):
        pltpu.sync_copy(zbuf, acc_sp.at[pl.ds(base + jj * RCH, RCH)])
    plsc.subcore_barrier()
    @pl.loop(0, NECH)
    def _(j):
        pltpu.sync_copy(gbuf, acc_sp.at[col_v.at[j]], add=True)
    plsc.subcore_barrier()

    t0v = tempv[0, :]
    @pl.loop(0, NCH)
    def _(jj):
        r0 = base + jj * RCH
        pltpu.sync_copy(acc_sp.at[pl.ds(r0, RCH)], abuf)
        pltpu.sync_copy(x_hbm.at[pl.ds(cnp + r0, RCH)], ubuf)
        @pl.loop(0, RCH)
        def _(i):
            d = abuf[i, pl.ds(0, 16)] + ones
            d2 = ones / d
            dis2b[jj * RCH + i, :] = d2
            dv = ones / babylonian_sqrt(d)
            for g in range(HD // 16):
                sl = pl.ds(g * 16, 16)
                un = dv * ubuf[i, sl]
                ubuf[i, sl] = un
                sbuf[i, sl] = t0v * un
        pltpu.sync_copy(ubuf, u_hbm.at[pl.ds(cnp + r0, RCH)])
        pltpu.sync_copy(sbuf, s_hbm.at[pl.ds(cnp + r0, RCH)])
        pltpu.sync_copy(zbuf, acc_sp.at[pl.ds(r0, RCH)])
    plsc.subcore_barrier()

    for k in range(K):
        @pl.loop(0, NECH)
        def _(j):
            pltpu.sync_copy(u_hbm.at[row_v.at[j]], gbuf)
            pltpu.sync_copy(gbuf, acc_sp.at[col_v.at[j]], add=True)
        plsc.subcore_barrier()

        tkv = tempv[k + 1, :]
        @pl.loop(0, NCH)
        def _(jj):
            r0 = base + jj * RCH
            pltpu.sync_copy(acc_sp.at[pl.ds(r0, RCH)], abuf)
            pltpu.sync_copy(u_hbm.at[pl.ds(cnp + r0, RCH)], ubuf)
            pltpu.sync_copy(s_hbm.at[pl.ds(cnp + r0, RCH)], sbuf)
            @pl.loop(0, RCH)
            def _(i):
                d2 = dis2b[jj * RCH + i, :]
                for g in range(HD // 16):
                    sl = pl.ds(g * 16, 16)
                    un = d2 * (abuf[i, sl] + ubuf[i, sl])
                    ubuf[i, sl] = un
                    sbuf[i, sl] = sbuf[i, sl] + tkv * un
            pltpu.sync_copy(ubuf, u_hbm.at[pl.ds(cnp + r0, RCH)])
            pltpu.sync_copy(sbuf, s_hbm.at[pl.ds(cnp + r0, RCH)])
            pltpu.sync_copy(zbuf, acc_sp.at[pl.ds(r0, RCH)])
        plsc.subcore_barrier()

    @pl.loop(0, NCH)
    def _(jj):
        r0 = base + jj * RCH
        pltpu.sync_copy(s_hbm.at[pl.ds(cnp + r0, RCH)], sbuf)
        @pl.loop(0, RCH)
        def _(i):
            d2 = dis2b[jj * RCH + i, :]
            iv = babylonian_sqrt(ones / d2)
            for g in range(HD // 16):
                sl = pl.ds(g * 16, 16)
                sbuf[i, sl] = iv * sbuf[i, sl]
        pltpu.sync_copy(sbuf, hid_hbm.at[pl.ds(cnp + r0, RCH)])


def _propagate(x_flat, rowp, colp, temp_b):
    mesh = plsc.VectorSubcoreMesh(core_axis_name="c", subcore_axis_name="s")
    f32 = jnp.float32
    kfn = pl.kernel(
        _sc_body,
        out_type=[
            jax.ShapeDtypeStruct((2 * NP, HD), f32),
            jax.ShapeDtypeStruct((2 * NP, HD), f32),
            jax.ShapeDtypeStruct((2 * NP, HD), f32),
        ],
        mesh=mesh,
        compiler_params=pltpu.CompilerParams(use_tc_tiling_on_sc=False),
        scratch_types=[
            pltpu.VMEM((NECH, ECH), jnp.int32),
            pltpu.VMEM((NECH, ECH), jnp.int32),
            pltpu.VMEM((ECH, HD), f32),
            pltpu.VMEM((RCH, HD), f32),
            pltpu.VMEM((RCH, HD), f32),
            pltpu.VMEM((RCH, HD), f32),
            pltpu.VMEM((RCH, HD), f32),
            pltpu.VMEM((ROWS_PER_TILE, 16), f32),
            pltpu.VMEM((16, 16), f32),
            pltpu.VMEM_SHARED((NP, HD), f32),
        ],
    )
    hid, _, _ = kfn(x_flat, rowp, colp, temp_b)
    return hid


def _mlp_body(h_ref, w1_ref, b1_ref, w2_ref, b2_ref, o_ref):
    z = jnp.dot(h_ref[...], w1_ref[...], preferred_element_type=jnp.float32)
    z = jnp.maximum(z + b1_ref[...], 0.0)
    lg = jnp.dot(z, w2_ref[...], preferred_element_type=jnp.float32)
    lg = lg + b2_ref[...]
    m = jnp.max(lg, axis=1, keepdims=True)
    s = jnp.log(jnp.sum(jnp.exp(lg - m), axis=1, keepdims=True))
    o_ref[...] = lg - m - s


def _mlp(hidden, W1, b1, W2, b2):
    BN = 1000
    grid = (N // BN,)
    return pl.pallas_call(
        _mlp_body,
        grid=grid,
        in_specs=[
            pl.BlockSpec((BN, D), lambda i: (i, 0)),
            pl.BlockSpec((D, H), lambda i: (0, 0)),
            pl.BlockSpec((1, H), lambda i: (0, 0)),
            pl.BlockSpec((H, C), lambda i: (0, 0)),
            pl.BlockSpec((1, C), lambda i: (0, 0)),
        ],
        out_specs=pl.BlockSpec((BN, C), lambda i: (i, 0)),
        out_shape=jax.ShapeDtypeStruct((N, C), jnp.float32),
    )(hidden, W1, b1.reshape(1, H), W2, b2.reshape(1, C))


@jax.jit
def kernel(x, edge_index, temp, W1, b1, W2, b2):
    row = edge_index[0]
    col = edge_index[1]
    pad = 16 * EPT - E
    rowp = jnp.concatenate([row, jnp.zeros((pad,), jnp.int32)])
    colp = jnp.concatenate([col, jnp.full((pad,), N, jnp.int32)])
    rowp = rowp.reshape(16, NECH, ECH)
    colp = colp.reshape(16, NECH, ECH)
    x0 = jnp.pad(x[:, :HD], ((0, NP - N), (0, 0)))
    x1 = jnp.pad(x[:, HD:], ((0, NP - N), (0, 0)))
    x_flat = jnp.concatenate([x0, x1], axis=0)
    temp_b = jnp.broadcast_to(jnp.pad(temp, (0, 16 - (K + 1)))[:, None],
                              (16, 16)).astype(jnp.float32)
    hid = _propagate(x_flat, rowp, colp, temp_b)
    hidden = jnp.concatenate([hid[:N], hid[NP:NP + N]], axis=1)
    return _mlp(hidden, W1, b1, W2, b2)


# speedup vs baseline: 7.6180x; 1.3111x over previous
import functools

import jax
import jax.numpy as jnp
from jax import lax
from jax.experimental import pallas as pl
from jax.experimental.pallas import tpu as pltpu
from jax.experimental.pallas import tpu_sc as plsc

N = 10000
E = 320000
D = 128
H = 64
C = 40
K = 10

NP = 10240
ROWS_PER_TILE = NP // 16
RCH = 64
NCH = ROWS_PER_TILE // RCH
EPT = 20480
ECH = 128
NECH = EPT // ECH
HD = D // 2


def _zero_rows(ref, nrows):
    z = jnp.zeros((16,), jnp.float32)
    @pl.loop(0, nrows)
    def _(i):
        for g in range(HD // 16):
            ref[i, pl.ds(g * 16, 16)] = z


def _sc_body(x_hbm, rowp_hbm, colp_hbm, temp_hbm,
             hid_hbm, u_hbm, s_hbm,
             row_v, col_v, gbuf, gbuf2, abuf, ubuf, sbuf, zbuf,
             dis2b, tempv, gsem, gsem2,
             acc_sp):
    c = lax.axis_index("c")
    tid = lax.axis_index("s")
    cnp = (c * NP).astype(jnp.int32)
    base = tid * ROWS_PER_TILE

    ones = jnp.full((16,), 1.0, jnp.float32)
    half = jnp.full((16,), 0.5, jnp.float32)

    def babylonian_sqrt(d):
        y = half * (ones + d)
        for _it in range(12):
            y = half * (y + d / y)
        return y

    pltpu.sync_copy(rowp_hbm.at[tid], row_v)
    pltpu.sync_copy(colp_hbm.at[tid], col_v)
    pltpu.sync_copy(temp_hbm, tempv)
    cnp_v = jnp.full((16,), cnp, jnp.int32)
    @pl.loop(0, NECH)
    def _(j):
        for g in range(ECH // 16):
            sl = pl.ds(g * 16, 16)
            row_v[j, sl] = row_v[j, sl] + cnp_v

    _zero_rows(zbuf, RCH)

    @pl.loop(0, ECH)
    def _(i):
        for g in range(HD // 16):
            gbuf[i, pl.ds(g * 16, 16)] = ones
    @pl.loop(0, NCH)
    def _(jj):
        pltpu.sync_copy(zbuf, acc_sp.at[pl.ds(base + jj * RCH, RCH)])
    plsc.subcore_barrier()
    @pl.loop(0, NECH)
    def _(j):
        pltpu.sync_copy(gbuf, acc_sp.at[col_v.at[j]], add=True)
    plsc.subcore_barrier()

    t0v = tempv[0, :]
    @pl.loop(0, NCH)
    def _(jj):
        r0 = base + jj * RCH
        pltpu.sync_copy(acc_sp.at[pl.ds(r0, RCH)], abuf)
        pltpu.sync_copy(x_hbm.at[pl.ds(cnp + r0, RCH)], ubuf)
        @pl.loop(0, RCH)
        def _(i):
            d = abuf[i, pl.ds(0, 16)] + ones
            d2 = ones / d
            dis2b[jj * RCH + i, :] = d2
            dv = ones / babylonian_sqrt(d)
            for g in range(HD // 16):
                sl = pl.ds(g * 16, 16)
                un = dv * ubuf[i, sl]
                ubuf[i, sl] = un
                sbuf[i, sl] = t0v * un
        pltpu.sync_copy(ubuf, u_hbm.at[pl.ds(cnp + r0, RCH)])
        pltpu.sync_copy(sbuf, s_hbm.at[pl.ds(cnp + r0, RCH)])
        pltpu.sync_copy(zbuf, acc_sp.at[pl.ds(r0, RCH)])
    plsc.subcore_barrier()

    def _gather_start(j, buf, sem):
        return pltpu.async_copy(u_hbm.at[row_v.at[j]], buf, sem)

    def _gather_wait(buf, sem):
        pltpu.make_async_copy(u_hbm.at[row_v.at[0]], buf, sem).wait()

    for k in range(K):
        _gather_start(0, gbuf, gsem)
        @pl.loop(0, NECH // 2 - 1)
        def _(j2):
            b = 2 * j2
            _gather_start(b + 1, gbuf2, gsem2)
            _gather_wait(gbuf, gsem)
            pltpu.sync_copy(gbuf, acc_sp.at[col_v.at[b]], add=True)
            _gather_start(b + 2, gbuf, gsem)
            _gather_wait(gbuf2, gsem2)
            pltpu.sync_copy(gbuf2, acc_sp.at[col_v.at[b + 1]], add=True)
        _gather_start(NECH - 1, gbuf2, gsem2)
        _gather_wait(gbuf, gsem)
        pltpu.sync_copy(gbuf, acc_sp.at[col_v.at[NECH - 2]], add=True)
        _gather_wait(gbuf2, gsem2)
        pltpu.sync_copy(gbuf2, acc_sp.at[col_v.at[NECH - 1]], add=True)
        plsc.subcore_barrier()

        tkv = tempv[k + 1, :]
        @pl.loop(0, NCH)
        def _(jj):
            r0 = base + jj * RCH
            pltpu.sync_copy(acc_sp.at[pl.ds(r0, RCH)], abuf)
            pltpu.sync_copy(u_hbm.at[pl.ds(cnp + r0, RCH)], ubuf)
            pltpu.sync_copy(s_hbm.at[pl.ds(cnp + r0, RCH)], sbuf)
            @pl.loop(0, RCH)
            def _(i):
                d2 = dis2b[jj * RCH + i, :]
                for g in range(HD // 16):
                    sl = pl.ds(g * 16, 16)
                    un = d2 * (abuf[i, sl] + ubuf[i, sl])
                    ubuf[i, sl] = un
                    sbuf[i, sl] = sbuf[i, sl] + tkv * un
            pltpu.sync_copy(ubuf, u_hbm.at[pl.ds(cnp + r0, RCH)])
            pltpu.sync_copy(sbuf, s_hbm.at[pl.ds(cnp + r0, RCH)])
            pltpu.sync_copy(zbuf, acc_sp.at[pl.ds(r0, RCH)])
        plsc.subcore_barrier()

    @pl.loop(0, NCH)
    def _(jj):
        r0 = base + jj * RCH
        pltpu.sync_copy(s_hbm.at[pl.ds(cnp + r0, RCH)], sbuf)
        @pl.loop(0, RCH)
        def _(i):
            d2 = dis2b[jj * RCH + i, :]
            iv = babylonian_sqrt(ones / d2)
            for g in range(HD // 16):
                sl = pl.ds(g * 16, 16)
                sbuf[i, sl] = iv * sbuf[i, sl]
        pltpu.sync_copy(sbuf, hid_hbm.at[pl.ds(cnp + r0, RCH)])


def _propagate(x_flat, rowp, colp, temp_b):
    mesh = plsc.VectorSubcoreMesh(core_axis_name="c", subcore_axis_name="s")
    f32 = jnp.float32
    kfn = pl.kernel(
        _sc_body,
        out_type=[
            jax.ShapeDtypeStruct((2 * NP, HD), f32),
            jax.ShapeDtypeStruct((2 * NP, HD), f32),
            jax.ShapeDtypeStruct((2 * NP, HD), f32),
        ],
        mesh=mesh,
        compiler_params=pltpu.CompilerParams(use_tc_tiling_on_sc=False),
        scratch_types=[
            pltpu.VMEM((NECH, ECH), jnp.int32),
            pltpu.VMEM((NECH, ECH), jnp.int32),
            pltpu.VMEM((ECH, HD), f32),
            pltpu.VMEM((ECH, HD), f32),
            pltpu.VMEM((RCH, HD), f32),
            pltpu.VMEM((RCH, HD), f32),
            pltpu.VMEM((RCH, HD), f32),
            pltpu.VMEM((RCH, HD), f32),
            pltpu.VMEM((ROWS_PER_TILE, 16), f32),
            pltpu.VMEM((16, 16), f32),
            pltpu.SemaphoreType.DMA,
            pltpu.SemaphoreType.DMA,
            pltpu.VMEM_SHARED((NP, HD), f32),
        ],
    )
    hid, _, _ = kfn(x_flat, rowp, colp, temp_b)
    return hid


def _mlp_body(h_ref, w1_ref, b1_ref, w2_ref, b2_ref, o_ref):
    z = jnp.dot(h_ref[...], w1_ref[...], preferred_element_type=jnp.float32)
    z = jnp.maximum(z + b1_ref[...], 0.0)
    lg = jnp.dot(z, w2_ref[...], preferred_element_type=jnp.float32)
    lg = lg + b2_ref[...]
    m = jnp.max(lg, axis=1, keepdims=True)
    s = jnp.log(jnp.sum(jnp.exp(lg - m), axis=1, keepdims=True))
    o_ref[...] = lg - m - s


def _mlp(hidden, W1, b1, W2, b2):
    BN = 1000
    grid = (N // BN,)
    return pl.pallas_call(
        _mlp_body,
        grid=grid,
        in_specs=[
            pl.BlockSpec((BN, D), lambda i: (i, 0)),
            pl.BlockSpec((D, H), lambda i: (0, 0)),
            pl.BlockSpec((1, H), lambda i: (0, 0)),
            pl.BlockSpec((H, C), lambda i: (0, 0)),
            pl.BlockSpec((1, C), lambda i: (0, 0)),
        ],
        out_specs=pl.BlockSpec((BN, C), lambda i: (i, 0)),
        out_shape=jax.ShapeDtypeStruct((N, C), jnp.float32),
    )(hidden, W1, b1.reshape(1, H), W2, b2.reshape(1, C))


@jax.jit
def kernel(x, edge_index, temp, W1, b1, W2, b2):
    row = edge_index[0]
    col = edge_index[1]
    pad = 16 * EPT - E
    rowp = jnp.concatenate([row, jnp.zeros((pad,), jnp.int32)])
    colp = jnp.concatenate([col, jnp.full((pad,), N, jnp.int32)])
    rowp = rowp.reshape(16, NECH, ECH)
    colp = colp.reshape(16, NECH, ECH)
    x0 = jnp.pad(x[:, :HD], ((0, NP - N), (0, 0)))
    x1 = jnp.pad(x[:, HD:], ((0, NP - N), (0, 0)))
    x_flat = jnp.concatenate([x0, x1], axis=0)
    temp_b = jnp.broadcast_to(jnp.pad(temp, (0, 16 - (K + 1)))[:, None],
                              (16, 16)).astype(jnp.float32)
    hid = _propagate(x_flat, rowp, colp, temp_b)
    hidden = jnp.concatenate([hid[:N], hid[NP:NP + N]], axis=1)
    return _mlp(hidden, W1, b1, W2, b2)
